# baseline (speedup 1.0000x reference)
import jax
import jax.numpy as jnp
from jax import lax
from jax.experimental import pallas as pl
from jax.experimental.pallas import tpu as pltpu, tpu_sc as plsc

_NC, _NS = 2, 16
_NW = _NC * _NS


def _make_sc_gather(S, W, D):
    CB = D // 8
    KS = 256
    PITCH = KS + 1
    nsblk = S // KS
    npair = W * nsblk
    per_w = npair // _NW
    sb_tot = S // 128
    assert per_w * _NW == npair and per_w >= 3
    mesh = plsc.VectorSubcoreMesh(
        core_axis_name="c", subcore_axis_name="s",
        num_cores=_NC, num_subcores=_NS)

    def body(idx_hbm, table_hbm, out_hbm, idx_v, rows_v, skew_v,
             isem0, isem1, gsem0, gsem1, wsem0, wsem1):
        isem, gsem, wsem = (isem0, isem1), (gsem0, gsem1), (wsem0, wsem1)
        wid = lax.axis_index("s") * _NC + lax.axis_index("c")
        p0 = wid * per_w
        pend = p0 + per_w

        lane = lax.iota(jnp.int32, 16)

        def i_copy(p, b):
            w, sblk = p // nsblk, lax.rem(p, nsblk)
            return pltpu.make_async_copy(
                idx_hbm.at[pl.ds(w * S + sblk * KS, KS)], idx_v.at[b], isem[b])

        def g_copy(b):
            return pltpu.make_async_copy(
                table_hbm.at[idx_v.at[b]], rows_v.at[b], gsem[b])

        def w_copy(p, b, cb, t):
            w, sblk = p // nsblk, lax.rem(p, nsblk)
            tile = w * (CB * sb_tot) + cb * sb_tot + sblk * (KS // 128) + t
            return pltpu.make_async_copy(
                skew_v.at[b, pl.ds(cb * 8, 8), pl.ds(t * 128, 128)],
                out_hbm.at[tile], wsem[b])

        def transpose(b):
            @pl.loop(0, KS, unroll=4)
            def _(i):
                iv = jnp.broadcast_to(i, (16,))
                for j in range(D // 16):
                    vals = rows_v[b, i, pl.ds(j * 16, 16)]
                    plsc.store_scatter(skew_v.at[b], [lane + (j * 16), iv],
                                       vals)

        def _step(p, b, nb):
            g_copy(b).wait()

            @pl.when(p + 1 < pend)
            def _():
                i_copy(p + 1, nb).wait()
                g_copy(nb).start()

                @pl.when(p + 2 < pend)
                def _():
                    i_copy(p + 2, b).start()

            @pl.when(p - 2 >= p0)
            def _():
                for cb in range(CB):
                    for t in range(KS // 128):
                        w_copy(p - 2, b, cb, t).wait()

            transpose(b)
            for cb in range(CB):
                for t in range(KS // 128):
                    w_copy(p, b, cb, t).start()

        i_copy(p0, 0).start()
        i_copy(p0 + 1, 1).start()
        i_copy(p0, 0).wait()
        g_copy(0).start()

        @pl.loop(0, per_w)
        def _(k):
            p = p0 + k

            @pl.when(lax.rem(k, 2) == 0)
            def _():
                _step(p, 0, 1)

            @pl.when(lax.rem(k, 2) == 1)
            def _():
                _step(p, 1, 0)

        b_last = (per_w - 1) % 2
        for cb in range(CB):
            for t in range(KS // 128):
                w_copy(pend - 2, 1 - b_last, cb, t).wait()
        for cb in range(CB):
            for t in range(KS // 128):
                w_copy(pend - 1, b_last, cb, t).wait()

    return pl.kernel(
        body,
        out_type=jax.ShapeDtypeStruct((W * CB * sb_tot, 8, 128), jnp.float32),
        mesh=mesh,
        compiler_params=pltpu.CompilerParams(
            use_tc_tiling_on_sc=False, needs_layout_passes=False),
        scratch_types=[
            pltpu.VMEM((2, KS), jnp.int32),
            pltpu.VMEM((2, KS, D), jnp.float32),
            pltpu.VMEM((2, D, PITCH), jnp.float32),
        ] + [pltpu.SemaphoreType.DMA] * 6,
    )


def kernel(inputs, table):
    s, w = inputs.shape
    _, d = table.shape
    table2 = jnp.repeat(table, 2, axis=0)
    idx_t = (jnp.transpose(inputs) * 2).reshape(w * s).astype(jnp.int32)
    tiles = _make_sc_gather(s, w, d)(idx_t, table2)
    out5 = tiles.reshape(w, d // 8, s // 128, 8, 128)
    return out5.transpose(2, 4, 0, 1, 3).reshape(s, w, d)


# speedup vs baseline: 1.8710x; 1.8710x over previous
import jax
import jax.numpy as jnp
from jax import lax
from jax.experimental import pallas as pl
from jax.experimental.pallas import tpu as pltpu, tpu_sc as plsc

_NC, _NS = 2, 16
_NW = _NC * _NS


def _make_sc_gather(S, W, D):
    CB = D // 8
    KS = 256
    PITCH = KS + 1
    nsblk = S // KS
    npair = W * nsblk
    per_w = npair // _NW
    sb_tot = S // 128
    assert per_w * _NW == npair and per_w >= 3
    mesh = plsc.VectorSubcoreMesh(
        core_axis_name="c", subcore_axis_name="s",
        num_cores=_NC, num_subcores=_NS)

    def body(idx_hbm, table_hbm, out_hbm, idx_v, rows_v, skew_v,
             isem0, isem1, gsem0, gsem1, wsem0, wsem1):
        isem, gsem, wsem = (isem0, isem1), (gsem0, gsem1), (wsem0, wsem1)
        wid = lax.axis_index("s") * _NC + lax.axis_index("c")
        p0 = wid * per_w
        pend = p0 + per_w

        lane = lax.iota(jnp.int32, 16)

        def i_copy(p, b):
            w, sblk = p // nsblk, lax.rem(p, nsblk)
            return pltpu.make_async_copy(
                idx_hbm.at[pl.ds(w * S + sblk * KS, KS)], idx_v.at[b], isem[b])

        def g_copy(b):
            return pltpu.make_async_copy(
                table_hbm.at[idx_v.at[b]], rows_v.at[b], gsem[b])

        def w_copy(p, b, cb, t):
            w, sblk = p // nsblk, lax.rem(p, nsblk)
            tile = w * (CB * sb_tot) + cb * sb_tot + sblk * (KS // 128) + t
            return pltpu.make_async_copy(
                skew_v.at[b, pl.ds(cb * 8, 8), pl.ds(t * 128, 128)],
                out_hbm.at[tile], wsem[b])

        def transpose(b):
            @pl.loop(0, KS, unroll=4)
            def _(i):
                iv = jnp.broadcast_to(i, (16,))
                for j in range(D // 16):
                    vals = rows_v[b, i, pl.ds(j * 16, 16)]
                    plsc.store_scatter(skew_v.at[b], [lane + (j * 16), iv],
                                       vals)

        def _step(p, b, nb):
            g_copy(b).wait()

            @pl.when(p + 1 < pend)
            def _():
                i_copy(p + 1, nb).wait()
                g_copy(nb).start()

                @pl.when(p + 2 < pend)
                def _():
                    i_copy(p + 2, b).start()

            @pl.when(p - 2 >= p0)
            def _():
                for cb in range(CB):
                    for t in range(KS // 128):
                        w_copy(p - 2, b, cb, t).wait()

            transpose(b)
            for cb in range(CB):
                for t in range(KS // 128):
                    w_copy(p, b, cb, t).start()

        i_copy(p0, 0).start()
        i_copy(p0 + 1, 1).start()
        i_copy(p0, 0).wait()
        g_copy(0).start()

        @pl.loop(0, per_w)
        def _(k):
            p = p0 + k

            @pl.when(lax.rem(k, 2) == 0)
            def _():
                _step(p, 0, 1)

            @pl.when(lax.rem(k, 2) == 1)
            def _():
                _step(p, 1, 0)

        b_last = (per_w - 1) % 2
        for cb in range(CB):
            for t in range(KS // 128):
                w_copy(pend - 2, 1 - b_last, cb, t).wait()
        for cb in range(CB):
            for t in range(KS // 128):
                w_copy(pend - 1, b_last, cb, t).wait()

    return pl.kernel(
        body,
        out_type=jax.ShapeDtypeStruct((W * CB * sb_tot, 8, 128), jnp.float32),
        mesh=mesh,
        compiler_params=pltpu.CompilerParams(
            use_tc_tiling_on_sc=False, needs_layout_passes=False),
        scratch_types=[
            pltpu.VMEM((2, KS), jnp.int32),
            pltpu.VMEM((2, KS, D), jnp.float32),
            pltpu.VMEM((2, D, PITCH), jnp.float32),
        ] + [pltpu.SemaphoreType.DMA] * 6,
    )


def kernel(inputs, table):
    s, w = inputs.shape
    _, d = table.shape
    idx_t = jnp.transpose(inputs).reshape(w * s).astype(jnp.int32)
    tiles = _make_sc_gather(s, w, d)(idx_t, table)
    out5 = tiles.reshape(w, d // 8, s // 128, 8, 128)
    return out5.transpose(2, 4, 0, 1, 3).reshape(s, w, d)


# speedup vs baseline: 1.8831x; 1.0065x over previous
import jax
import jax.numpy as jnp
from jax import lax
from jax.experimental import pallas as pl
from jax.experimental.pallas import tpu as pltpu, tpu_sc as plsc

_NC, _NS = 2, 16
_NW = _NC * _NS


def _make_sc_gather(S, W, D):
    CB = D // 8
    KS = 256
    PITCH = KS + 1
    nsblk = S // KS
    npair = W * nsblk
    per_w = npair // _NW
    sb_tot = S // 128
    assert per_w * _NW == npair and per_w >= 3
    mesh = plsc.VectorSubcoreMesh(
        core_axis_name="c", subcore_axis_name="s",
        num_cores=_NC, num_subcores=_NS)

    def body(idx_hbm, table_hbm, out_hbm, idx_v, rows_v, skew_v,
             isem0, isem1, gsem0, gsem1, wsem0, wsem1):
        isem, gsem, wsem = (isem0, isem1), (gsem0, gsem1), (wsem0, wsem1)
        wid = lax.axis_index("s") * _NC + lax.axis_index("c")
        p0 = wid * per_w
        pend = p0 + per_w

        lane = lax.iota(jnp.int32, 16)

        def i_copy(p, b):
            w, sblk = p // nsblk, lax.rem(p, nsblk)
            return pltpu.make_async_copy(
                idx_hbm.at[pl.ds(w * S + sblk * KS, KS)], idx_v.at[b], isem[b])

        def g_copy(b):
            return pltpu.make_async_copy(
                table_hbm.at[idx_v.at[b]], rows_v.at[b], gsem[b])

        def w_copy(p, b, cb, t):
            w, sblk = p // nsblk, lax.rem(p, nsblk)
            tile = w * (CB * sb_tot) + cb * sb_tot + sblk * (KS // 128) + t
            return pltpu.make_async_copy(
                skew_v.at[b, pl.ds(cb * 8, 8), pl.ds(t * 128, 128)],
                out_hbm.at[tile], wsem[b])

        def transpose(b):
            @pl.loop(0, KS, unroll=8)
            def _(i):
                iv = jnp.broadcast_to(i, (16,))
                for j in range(D // 16):
                    vals = rows_v[b, i, pl.ds(j * 16, 16)]
                    plsc.store_scatter(skew_v.at[b], [lane + (j * 16), iv],
                                       vals)

        def _step(p, b, nb):
            g_copy(b).wait()

            @pl.when(p + 1 < pend)
            def _():
                i_copy(p + 1, nb).wait()
                g_copy(nb).start()

                @pl.when(p + 2 < pend)
                def _():
                    i_copy(p + 2, b).start()

            @pl.when(p - 2 >= p0)
            def _():
                for cb in range(CB):
                    for t in range(KS // 128):
                        w_copy(p - 2, b, cb, t).wait()

            transpose(b)
            for cb in range(CB):
                for t in range(KS // 128):
                    w_copy(p, b, cb, t).start()

        i_copy(p0, 0).start()
        i_copy(p0 + 1, 1).start()
        i_copy(p0, 0).wait()
        g_copy(0).start()

        @pl.loop(0, per_w)
        def _(k):
            p = p0 + k

            @pl.when(lax.rem(k, 2) == 0)
            def _():
                _step(p, 0, 1)

            @pl.when(lax.rem(k, 2) == 1)
            def _():
                _step(p, 1, 0)

        b_last = (per_w - 1) % 2
        for cb in range(CB):
            for t in range(KS // 128):
                w_copy(pend - 2, 1 - b_last, cb, t).wait()
        for cb in range(CB):
            for t in range(KS // 128):
                w_copy(pend - 1, b_last, cb, t).wait()

    return pl.kernel(
        body,
        out_type=jax.ShapeDtypeStruct((W * CB * sb_tot, 8, 128), jnp.float32),
        mesh=mesh,
        compiler_params=pltpu.CompilerParams(
            use_tc_tiling_on_sc=False, needs_layout_passes=False),
        scratch_types=[
            pltpu.VMEM((2, KS), jnp.int32),
            pltpu.VMEM((2, KS, D), jnp.float32),
            pltpu.VMEM((2, D, PITCH), jnp.float32),
        ] + [pltpu.SemaphoreType.DMA] * 6,
    )


def kernel(inputs, table):
    s, w = inputs.shape
    _, d = table.shape
    idx_t = jnp.transpose(inputs).reshape(w * s).astype(jnp.int32)
    tiles = _make_sc_gather(s, w, d)(idx_t, table)
    out5 = tiles.reshape(w, d // 8, s // 128, 8, 128)
    return out5.transpose(2, 4, 0, 1, 3).reshape(s, w, d)
